# baseline (speedup 1.0000x reference)
import functools
import math

import jax
import jax.numpy as jnp
from jax import lax
from jax.experimental import pallas as pl
from jax.experimental.pallas import tpu as pltpu
from jax.experimental.pallas import tpu_sc as plsc

_NCOLS = 2048
_CHUNK_ROWS = 32
_CHUNK_WORDS = _CHUNK_ROWS * _NCOLS
_NUM_WORKERS = 32
_PPAD = 30720
_PPW = _PPAD // _NUM_WORKERS
_M = 8192
_K = 4096


def _sc_body(ep_h, ovl_h, qr_h, qs_h, pr_h, ps_h, cst_h,
             c_out_h, f_out_h,
             ep_v, ovl_v, elin_v, qr_v, qs_v, qlin_v, mapb,
             pr_v, ps_v, cst_v, acc_c_v, acc_f_v,
             sem_a, sem_b, sem_c):
    wid = lax.axis_index("s") * 2 + lax.axis_index("c")

    pbase = wid * (_PPW * 3)
    cp_pts_r = pltpu.async_copy(pr_h.at[pl.ds(pbase, _PPW * 3)], pr_v, sem_a)
    cp_pts_s = pltpu.async_copy(ps_h.at[pl.ds(pbase, _PPW * 3)], ps_v, sem_a)
    cp_cst = pltpu.async_copy(cst_h, cst_v, sem_a)
    cp_ep = pltpu.async_copy(ep_h, ep_v, sem_b)
    cp_ovl = pltpu.async_copy(ovl_h, ovl_v, sem_b)
    cp_qr = pltpu.async_copy(qr_h, qr_v, sem_c)
    cp_qs = pltpu.async_copy(qs_h, qs_v, sem_c)

    lane = lax.iota(jnp.int32, 16)
    zeros = jnp.zeros((16,), jnp.float32)
    ones = jnp.ones((16,), jnp.float32)

    cp_pts_r.wait()
    cp_pts_s.wait()
    cp_cst.wait()
    r00 = cst_v[0, :]; r01 = cst_v[1, :]; r02 = cst_v[2, :]
    r10 = cst_v[3, :]; r11 = cst_v[4, :]; r12 = cst_v[5, :]
    r20 = cst_v[6, :]; r21 = cst_v[7, :]; r22 = cst_v[8, :]
    t0 = cst_v[9, :]; t1 = cst_v[10, :]; t2 = cst_v[11, :]
    lane3 = lane * 3
    pidx0 = pbase // 3 + lane

    def fbody(i, acc):
        for j in range(4):
            o = (i * 4 + j) * 48
            ix = lane3 + o
            vx = plsc.load_gather(ps_v, [ix])
            vy = plsc.load_gather(ps_v, [ix + 1])
            vz = plsc.load_gather(ps_v, [ix + 2])
            tx = r00 * vx + r01 * vy + r02 * vz + t0
            ty = r10 * vx + r11 * vy + r12 * vz + t1
            tz = r20 * vx + r21 * vy + r22 * vz + t2
            dx = plsc.load_gather(pr_v, [ix]) - tx
            dy = plsc.load_gather(pr_v, [ix + 1]) - ty
            dz = plsc.load_gather(pr_v, [ix + 2]) - tz
            d2 = dx * dx + dy * dy + dz * dz
            gidx = pidx0 + (i * 4 + j) * 16
            m = (gidx < 30000) & (d2 < 0.01)
            acc = acc + jnp.where(m, 1.0, 0.0)
        return acc

    facc = lax.fori_loop(0, _PPW // 64, fbody, zeros)

    cp_ep.wait()
    cp_ovl.wait()
    lane2 = lane * 2

    def ebody(i, _):
        for j in range(4):
            o = i * 4 + j
            ix = lane2 + o * 32
            er = plsc.load_gather(ep_v, [ix])
            es = plsc.load_gather(ep_v, [ix + 1])
            eo = ovl_v[pl.ds(o * 16, 16)]
            lin = er * _NCOLS + es
            elin_v[pl.ds(o * 16, 16)] = jnp.where(eo > 0.0, lin, -1)
        return 0

    lax.fori_loop(0, _M // 64, ebody, 0)

    cp_qr.wait()
    cp_qs.wait()

    def qbody(i, _):
        for j in range(4):
            o = i * 4 + j
            qlin_v[pl.ds(o * 16, 16)] = (
                qr_v[pl.ds(o * 16, 16)] * _NCOLS + qs_v[pl.ds(o * 16, 16)])
        return 0

    lax.fori_loop(0, _K // 64, qbody, 0)

    nwords_u = jnp.uint32(_CHUNK_WORDS)
    cacc = zeros
    for half in range(2):
        base = (wid + _NUM_WORKERS * half) * _CHUNK_WORDS

        def zbody(i, _):
            for j in range(4):
                o = i * 4 + j
                rel = qlin_v[pl.ds(o * 16, 16)] - base
                m = plsc.bitcast(rel, jnp.uint32) < nwords_u
                idx = rel & (_CHUNK_WORDS - 1)
                plsc.store_scatter(mapb, [idx], zeros, mask=m)
            return 0

        lax.fori_loop(0, _K // 64, zbody, 0)

        def sbody(i, _):
            for j in range(4):
                o = i * 4 + j
                rel = elin_v[pl.ds(o * 16, 16)] - base
                m = plsc.bitcast(rel, jnp.uint32) < nwords_u
                idx = rel & (_CHUNK_WORDS - 1)
                plsc.store_scatter(mapb, [idx], ones, mask=m)
            return 0

        lax.fori_loop(0, _M // 64, sbody, 0)

        def gbody(i, acc):
            for j in range(4):
                o = i * 4 + j
                rel = qlin_v[pl.ds(o * 16, 16)] - base
                m = plsc.bitcast(rel, jnp.uint32) < nwords_u
                idx = rel & (_CHUNK_WORDS - 1)
                v = plsc.load_gather(mapb, [idx], mask=m)
                acc = acc + jnp.where(m, v, 0.0)
            return acc

        cacc = lax.fori_loop(0, _K // 64, gbody, cacc)

    acc_c_v[...] = cacc
    acc_f_v[...] = facc
    pltpu.sync_copy(acc_c_v, c_out_h.at[wid])
    pltpu.sync_copy(acc_f_v, f_out_h.at[wid])


def _tc_body(cpart_ref, fpart_ref, t_ref, e_ref, out_ref):
    c_prec = jnp.sum(cpart_ref[...]) * (1.0 / 4096.0)
    f_prec = jnp.sum(fpart_ref[...]) * (1.0 / 30000.0)
    t = t_ref[...]
    e = e_ref[...]
    tr = jnp.sum(t[:3, :3] * e[:3, :3])
    x = jnp.clip((tr - 1.0) * 0.5, -1.0, 1.0)
    acos = jnp.arctan2(jnp.sqrt(jnp.maximum(1.0 - x * x, 0.0)), x)
    rre = acos * (180.0 / math.pi)
    dt = t[:3, 3] - e[:3, 3]
    rte = jnp.sqrt(jnp.sum(dt * dt))
    recall = jnp.where((rre < 15.0) & (rte < 0.3), 1.0, 0.0)
    i8 = lax.broadcasted_iota(jnp.int32, (1, 8), 1)
    v = jnp.where(i8 == 0, c_prec,
        jnp.where(i8 == 1, f_prec,
        jnp.where(i8 == 2, rre,
        jnp.where(i8 == 3, rte,
        jnp.where(i8 == 4, recall, 0.0)))))
    out_ref[...] = v


def kernel(ref_points_c, src_points_c, gt_node_corr_overlaps, gt_node_corr_indices,
           ref_node_corr_indices, src_node_corr_indices, ref_corr_points,
           src_corr_points, transform, estimated_transform):
    ep = gt_node_corr_indices.astype(jnp.int32).reshape(-1)
    qr = ref_node_corr_indices.astype(jnp.int32)
    qs = src_node_corr_indices.astype(jnp.int32)
    p = ref_corr_points.shape[0]
    pr = jnp.pad(ref_corr_points, ((0, _PPAD - p), (0, 0))).reshape(-1)
    ps = jnp.pad(src_corr_points, ((0, _PPAD - p), (0, 0))).reshape(-1)
    cvals = jnp.concatenate([transform[:3, :3].reshape(-1), transform[:3, 3]])
    cst = jnp.broadcast_to(cvals[:, None], (12, 16)).astype(jnp.float32)

    mesh = plsc.VectorSubcoreMesh(core_axis_name="c", subcore_axis_name="s",
                                  num_cores=2, num_subcores=16)
    sc_fn = functools.partial(
        pl.kernel,
        out_type=[
            jax.ShapeDtypeStruct((_NUM_WORKERS, 16), jnp.float32),
            jax.ShapeDtypeStruct((_NUM_WORKERS, 16), jnp.float32),
        ],
        mesh=mesh,
        scratch_types=[
            pltpu.VMEM((2 * _M,), jnp.int32),
            pltpu.VMEM((_M,), jnp.float32),
            pltpu.VMEM((_M,), jnp.int32),
            pltpu.VMEM((_K,), jnp.int32),
            pltpu.VMEM((_K,), jnp.int32),
            pltpu.VMEM((_K,), jnp.int32),
            pltpu.VMEM((_CHUNK_WORDS,), jnp.float32),
            pltpu.VMEM((_PPW * 3,), jnp.float32),
            pltpu.VMEM((_PPW * 3,), jnp.float32),
            pltpu.VMEM((12, 16), jnp.float32),
            pltpu.VMEM((16,), jnp.float32),
            pltpu.VMEM((16,), jnp.float32),
            pltpu.SemaphoreType.DMA,
            pltpu.SemaphoreType.DMA,
            pltpu.SemaphoreType.DMA,
        ],
        compiler_params=pltpu.CompilerParams(needs_layout_passes=False),
    )(_sc_body)
    c_part, f_part = sc_fn(ep, gt_node_corr_overlaps, qr, qs, pr, ps, cst)

    res = pl.pallas_call(
        _tc_body,
        out_shape=jax.ShapeDtypeStruct((1, 8), jnp.float32),
    )(c_part, f_part, transform.astype(jnp.float32),
      estimated_transform.astype(jnp.float32))
    return res[0, :5]


# speedup vs baseline: 2.4443x; 2.4443x over previous
import functools
import math

import jax
import jax.numpy as jnp
from jax import lax
from jax.experimental import pallas as pl
from jax.experimental.pallas import tpu as pltpu
from jax.experimental.pallas import tpu_sc as plsc

_NCOLS = 2048
_CHUNK_ROWS = 32
_CHUNK_WORDS = _CHUNK_ROWS * _NCOLS
_NUM_WORKERS = 32
_PPAD = 30720
_PPW = _PPAD // _NUM_WORKERS
_M = 8192
_K = 4096


def _sc_body(gtt_h, ovl_h, qr_h, qs_h, prt_h, pst_h, cst_h,
             c_out_h, f_out_h,
             gr_v, gs_v, ovl_v, elin_v, qr_v, qs_v, qlin_v, mapb,
             rx_v, ry_v, rz_v, sx_v, sy_v, sz_v, cst_v, acc_c_v, acc_f_v,
             sem_a, sem_b, sem_c):
    wid = lax.axis_index("s") * 2 + lax.axis_index("c")

    pbase = wid * _PPW
    cp_rx = pltpu.async_copy(prt_h.at[0, pl.ds(pbase, _PPW)], rx_v, sem_a)
    cp_ry = pltpu.async_copy(prt_h.at[1, pl.ds(pbase, _PPW)], ry_v, sem_a)
    cp_rz = pltpu.async_copy(prt_h.at[2, pl.ds(pbase, _PPW)], rz_v, sem_a)
    cp_sx = pltpu.async_copy(pst_h.at[0, pl.ds(pbase, _PPW)], sx_v, sem_a)
    cp_sy = pltpu.async_copy(pst_h.at[1, pl.ds(pbase, _PPW)], sy_v, sem_a)
    cp_sz = pltpu.async_copy(pst_h.at[2, pl.ds(pbase, _PPW)], sz_v, sem_a)
    cp_cst = pltpu.async_copy(cst_h, cst_v, sem_a)
    cp_gr = pltpu.async_copy(gtt_h.at[0], gr_v, sem_b)
    cp_gs = pltpu.async_copy(gtt_h.at[1], gs_v, sem_b)
    cp_ovl = pltpu.async_copy(ovl_h, ovl_v, sem_b)
    cp_qr = pltpu.async_copy(qr_h, qr_v, sem_c)
    cp_qs = pltpu.async_copy(qs_h, qs_v, sem_c)

    lane = lax.iota(jnp.int32, 16)
    zeros = jnp.zeros((16,), jnp.float32)
    ones = jnp.ones((16,), jnp.float32)

    cp_rx.wait(); cp_ry.wait(); cp_rz.wait()
    cp_sx.wait(); cp_sy.wait(); cp_sz.wait()
    cp_cst.wait()
    r00 = cst_v[0, :]; r01 = cst_v[1, :]; r02 = cst_v[2, :]
    r10 = cst_v[3, :]; r11 = cst_v[4, :]; r12 = cst_v[5, :]
    r20 = cst_v[6, :]; r21 = cst_v[7, :]; r22 = cst_v[8, :]
    t0 = cst_v[9, :]; t1 = cst_v[10, :]; t2 = cst_v[11, :]
    pidx0 = pbase + lane

    def fbody(i, acc):
        for j in range(4):
            o = (i * 4 + j) * 16
            vx = sx_v[pl.ds(o, 16)]
            vy = sy_v[pl.ds(o, 16)]
            vz = sz_v[pl.ds(o, 16)]
            tx = r00 * vx + r01 * vy + r02 * vz + t0
            ty = r10 * vx + r11 * vy + r12 * vz + t1
            tz = r20 * vx + r21 * vy + r22 * vz + t2
            dx = rx_v[pl.ds(o, 16)] - tx
            dy = ry_v[pl.ds(o, 16)] - ty
            dz = rz_v[pl.ds(o, 16)] - tz
            d2 = dx * dx + dy * dy + dz * dz
            m = (pidx0 + o < 30000) & (d2 < 0.01)
            acc = acc + jnp.where(m, 1.0, 0.0)
        return acc

    facc = lax.fori_loop(0, _PPW // 64, fbody, zeros)

    cp_gr.wait(); cp_gs.wait(); cp_ovl.wait()

    def ebody(i, _):
        for j in range(4):
            o = (i * 4 + j) * 16
            er = gr_v[pl.ds(o, 16)]
            es = gs_v[pl.ds(o, 16)]
            eo = ovl_v[pl.ds(o, 16)]
            elin_v[pl.ds(o, 16)] = jnp.where(eo > 0.0, er * _NCOLS + es, -1)
        return 0

    lax.fori_loop(0, _M // 64, ebody, 0)

    cp_qr.wait(); cp_qs.wait()

    def qbody(i, _):
        for j in range(4):
            o = (i * 4 + j) * 16
            qlin_v[pl.ds(o, 16)] = qr_v[pl.ds(o, 16)] * _NCOLS + qs_v[pl.ds(o, 16)]
        return 0

    lax.fori_loop(0, _K // 64, qbody, 0)

    nwords_u = jnp.uint32(_CHUNK_WORDS)
    cacc = zeros
    for half in range(2):
        base = (wid + _NUM_WORKERS * half) * _CHUNK_WORDS

        def zbody(i, _):
            for j in range(4):
                o = (i * 4 + j) * 16
                rel = qlin_v[pl.ds(o, 16)] - base
                m = plsc.bitcast(rel, jnp.uint32) < nwords_u
                idx = rel & (_CHUNK_WORDS - 1)
                plsc.store_scatter(mapb, [idx], zeros, mask=m)
            return 0

        lax.fori_loop(0, _K // 64, zbody, 0)

        def sbody(i, _):
            for j in range(4):
                o = (i * 4 + j) * 16
                rel = elin_v[pl.ds(o, 16)] - base
                m = plsc.bitcast(rel, jnp.uint32) < nwords_u
                idx = rel & (_CHUNK_WORDS - 1)
                plsc.store_scatter(mapb, [idx], ones, mask=m)
            return 0

        lax.fori_loop(0, _M // 64, sbody, 0)

        def gbody(i, acc):
            for j in range(4):
                o = (i * 4 + j) * 16
                rel = qlin_v[pl.ds(o, 16)] - base
                m = plsc.bitcast(rel, jnp.uint32) < nwords_u
                idx = rel & (_CHUNK_WORDS - 1)
                v = plsc.load_gather(mapb, [idx], mask=m)
                acc = acc + jnp.where(m, v, 0.0)
            return acc

        cacc = lax.fori_loop(0, _K // 64, gbody, cacc)

    acc_c_v[...] = cacc
    acc_f_v[...] = facc
    pltpu.sync_copy(acc_c_v, c_out_h.at[wid])
    pltpu.sync_copy(acc_f_v, f_out_h.at[wid])


def _tc_body(cpart_ref, fpart_ref, t_ref, e_ref, out_ref):
    c_prec = jnp.sum(cpart_ref[...]) * (1.0 / 4096.0)
    f_prec = jnp.sum(fpart_ref[...]) * (1.0 / 30000.0)
    t = t_ref[...]
    e = e_ref[...]
    tr = jnp.sum(t[:3, :3] * e[:3, :3])
    x = jnp.clip((tr - 1.0) * 0.5, -1.0, 1.0)
    acos = jnp.arctan2(jnp.sqrt(jnp.maximum(1.0 - x * x, 0.0)), x)
    rre = acos * (180.0 / math.pi)
    dt = t[:3, 3] - e[:3, 3]
    rte = jnp.sqrt(jnp.sum(dt * dt))
    recall = jnp.where((rre < 15.0) & (rte < 0.3), 1.0, 0.0)
    i8 = lax.broadcasted_iota(jnp.int32, (1, 8), 1)
    v = jnp.where(i8 == 0, c_prec,
        jnp.where(i8 == 1, f_prec,
        jnp.where(i8 == 2, rre,
        jnp.where(i8 == 3, rte,
        jnp.where(i8 == 4, recall, 0.0)))))
    out_ref[...] = v


def kernel(ref_points_c, src_points_c, gt_node_corr_overlaps, gt_node_corr_indices,
           ref_node_corr_indices, src_node_corr_indices, ref_corr_points,
           src_corr_points, transform, estimated_transform):
    gtt = gt_node_corr_indices.astype(jnp.int32).T
    qr = ref_node_corr_indices.astype(jnp.int32)
    qs = src_node_corr_indices.astype(jnp.int32)
    p = ref_corr_points.shape[0]
    prt = jnp.pad(ref_corr_points.T, ((0, 0), (0, _PPAD - p)))
    pst = jnp.pad(src_corr_points.T, ((0, 0), (0, _PPAD - p)))
    cvals = jnp.concatenate([transform[:3, :3].reshape(-1), transform[:3, 3]])
    cst = jnp.broadcast_to(cvals[:, None], (12, 16)).astype(jnp.float32)

    mesh = plsc.VectorSubcoreMesh(core_axis_name="c", subcore_axis_name="s",
                                  num_cores=2, num_subcores=16)
    sc_fn = functools.partial(
        pl.kernel,
        out_type=[
            jax.ShapeDtypeStruct((_NUM_WORKERS, 16), jnp.float32),
            jax.ShapeDtypeStruct((_NUM_WORKERS, 16), jnp.float32),
        ],
        mesh=mesh,
        scratch_types=[
            pltpu.VMEM((_M,), jnp.int32),
            pltpu.VMEM((_M,), jnp.int32),
            pltpu.VMEM((_M,), jnp.float32),
            pltpu.VMEM((_M,), jnp.int32),
            pltpu.VMEM((_K,), jnp.int32),
            pltpu.VMEM((_K,), jnp.int32),
            pltpu.VMEM((_K,), jnp.int32),
            pltpu.VMEM((_CHUNK_WORDS,), jnp.float32),
            pltpu.VMEM((_PPW,), jnp.float32),
            pltpu.VMEM((_PPW,), jnp.float32),
            pltpu.VMEM((_PPW,), jnp.float32),
            pltpu.VMEM((_PPW,), jnp.float32),
            pltpu.VMEM((_PPW,), jnp.float32),
            pltpu.VMEM((_PPW,), jnp.float32),
            pltpu.VMEM((12, 16), jnp.float32),
            pltpu.VMEM((16,), jnp.float32),
            pltpu.VMEM((16,), jnp.float32),
            pltpu.SemaphoreType.DMA,
            pltpu.SemaphoreType.DMA,
            pltpu.SemaphoreType.DMA,
        ],
        compiler_params=pltpu.CompilerParams(needs_layout_passes=False,
                                             use_tc_tiling_on_sc=False),
    )(_sc_body)
    c_part, f_part = sc_fn(gtt, gt_node_corr_overlaps, qr, qs, prt, pst, cst)

    res = pl.pallas_call(
        _tc_body,
        out_shape=jax.ShapeDtypeStruct((1, 8), jnp.float32),
    )(c_part, f_part, transform.astype(jnp.float32),
      estimated_transform.astype(jnp.float32))
    return res[0, :5]


# speedup vs baseline: 2.7475x; 1.1241x over previous
import functools
import math

import jax
import jax.numpy as jnp
from jax import lax
from jax.experimental import pallas as pl
from jax.experimental.pallas import tpu as pltpu
from jax.experimental.pallas import tpu_sc as plsc

_NCOLS = 2048
_CHUNK_ROWS = 32
_CHUNK_WORDS = _CHUNK_ROWS * _NCOLS
_NUM_WORKERS = 32
_PPAD = 30720
_M = 8192
_K = 4096


def _sc_body(gtt_h, ovl_h, qr_h, qs_h,
             c_out_h,
             gr_v, gs_v, ovl_v, elin_v, qr_v, qs_v, qlin_v, mapb,
             acc_c_v, sem_b, sem_c):
    wid = lax.axis_index("s") * 2 + lax.axis_index("c")

    cp_gr = pltpu.async_copy(gtt_h.at[0], gr_v, sem_b)
    cp_gs = pltpu.async_copy(gtt_h.at[1], gs_v, sem_b)
    cp_ovl = pltpu.async_copy(ovl_h, ovl_v, sem_b)
    cp_qr = pltpu.async_copy(qr_h, qr_v, sem_c)
    cp_qs = pltpu.async_copy(qs_h, qs_v, sem_c)

    zeros = jnp.zeros((16,), jnp.float32)
    ones = jnp.ones((16,), jnp.float32)

    cp_gr.wait(); cp_gs.wait(); cp_ovl.wait()

    def ebody(i, _):
        for j in range(4):
            o = (i * 4 + j) * 16
            er = gr_v[pl.ds(o, 16)]
            es = gs_v[pl.ds(o, 16)]
            eo = ovl_v[pl.ds(o, 16)]
            elin_v[pl.ds(o, 16)] = jnp.where(eo > 0.0, er * _NCOLS + es, -1)
        return 0

    lax.fori_loop(0, _M // 64, ebody, 0)

    cp_qr.wait(); cp_qs.wait()

    def qbody(i, _):
        for j in range(4):
            o = (i * 4 + j) * 16
            qlin_v[pl.ds(o, 16)] = qr_v[pl.ds(o, 16)] * _NCOLS + qs_v[pl.ds(o, 16)]
        return 0

    lax.fori_loop(0, _K // 64, qbody, 0)

    nwords_u = jnp.uint32(_CHUNK_WORDS)
    cacc = zeros
    for half in range(2):
        base = (wid + _NUM_WORKERS * half) * _CHUNK_WORDS

        def zbody(i, _):
            for j in range(4):
                o = (i * 4 + j) * 16
                rel = qlin_v[pl.ds(o, 16)] - base
                m = plsc.bitcast(rel, jnp.uint32) < nwords_u
                idx = rel & (_CHUNK_WORDS - 1)
                plsc.store_scatter(mapb, [idx], zeros, mask=m)
            return 0

        lax.fori_loop(0, _K // 64, zbody, 0)

        def sbody(i, _):
            for j in range(4):
                o = (i * 4 + j) * 16
                rel = elin_v[pl.ds(o, 16)] - base
                m = plsc.bitcast(rel, jnp.uint32) < nwords_u
                idx = rel & (_CHUNK_WORDS - 1)
                plsc.store_scatter(mapb, [idx], ones, mask=m)
            return 0

        lax.fori_loop(0, _M // 64, sbody, 0)

        def gbody(i, acc):
            for j in range(4):
                o = (i * 4 + j) * 16
                rel = qlin_v[pl.ds(o, 16)] - base
                m = plsc.bitcast(rel, jnp.uint32) < nwords_u
                idx = rel & (_CHUNK_WORDS - 1)
                v = plsc.load_gather(mapb, [idx], mask=m)
                acc = acc + jnp.where(m, v, 0.0)
            return acc

        cacc = lax.fori_loop(0, _K // 64, gbody, cacc)

    acc_c_v[...] = cacc
    pltpu.sync_copy(acc_c_v, c_out_h.at[wid])


def _tc_fine_body(prt_ref, pst_ref, t_ref, e_ref, out_ref):
    t = t_ref[...]
    e = e_ref[...]
    pr = prt_ref[...]
    ps = pst_ref[...]
    sx = ps[0:1, :]
    sy = ps[1:2, :]
    sz = ps[2:3, :]
    dx = pr[0:1, :] - (t[0, 0] * sx + t[0, 1] * sy + t[0, 2] * sz + t[0, 3])
    dy = pr[1:2, :] - (t[1, 0] * sx + t[1, 1] * sy + t[1, 2] * sz + t[1, 3])
    dz = pr[2:3, :] - (t[2, 0] * sx + t[2, 1] * sy + t[2, 2] * sz + t[2, 3])
    d2 = dx * dx + dy * dy + dz * dz
    pidx = lax.broadcasted_iota(jnp.int32, (1, _PPAD), 1)
    inlier = (pidx < 30000) & (d2 < 0.01)
    f_prec = jnp.sum(jnp.where(inlier, 1.0, 0.0)) * (1.0 / 30000.0)
    tr = jnp.sum(t[:3, :3] * e[:3, :3])
    x = jnp.clip((tr - 1.0) * 0.5, -1.0, 1.0)
    acos = jnp.arctan2(jnp.sqrt(jnp.maximum(1.0 - x * x, 0.0)), x)
    rre = acos * (180.0 / math.pi)
    dt = t[:3, 3] - e[:3, 3]
    rte = jnp.sqrt(jnp.sum(dt * dt))
    recall = jnp.where((rre < 15.0) & (rte < 0.3), 1.0, 0.0)
    i8 = lax.broadcasted_iota(jnp.int32, (1, 8), 1)
    v = jnp.where(i8 == 1, f_prec,
        jnp.where(i8 == 2, rre,
        jnp.where(i8 == 3, rte,
        jnp.where(i8 == 4, recall, 0.0))))
    out_ref[...] = v


def _tc_combine_body(cpart_ref, fine_ref, out_ref):
    c_prec = jnp.sum(cpart_ref[...]) * (1.0 / 4096.0)
    i8 = lax.broadcasted_iota(jnp.int32, (1, 8), 1)
    out_ref[...] = jnp.where(i8 == 0, c_prec, fine_ref[...])


def kernel(ref_points_c, src_points_c, gt_node_corr_overlaps, gt_node_corr_indices,
           ref_node_corr_indices, src_node_corr_indices, ref_corr_points,
           src_corr_points, transform, estimated_transform):
    gtt = gt_node_corr_indices.astype(jnp.int32).T
    qr = ref_node_corr_indices.astype(jnp.int32)
    qs = src_node_corr_indices.astype(jnp.int32)
    p = ref_corr_points.shape[0]
    prt = jnp.pad(ref_corr_points.T, ((0, 0), (0, _PPAD - p)))
    pst = jnp.pad(src_corr_points.T, ((0, 0), (0, _PPAD - p)))

    fine = pl.pallas_call(
        _tc_fine_body,
        out_shape=jax.ShapeDtypeStruct((1, 8), jnp.float32),
    )(prt, pst, transform.astype(jnp.float32),
      estimated_transform.astype(jnp.float32))

    mesh = plsc.VectorSubcoreMesh(core_axis_name="c", subcore_axis_name="s",
                                  num_cores=2, num_subcores=16)
    sc_fn = functools.partial(
        pl.kernel,
        out_type=jax.ShapeDtypeStruct((_NUM_WORKERS, 16), jnp.float32),
        mesh=mesh,
        scratch_types=[
            pltpu.VMEM((_M,), jnp.int32),
            pltpu.VMEM((_M,), jnp.int32),
            pltpu.VMEM((_M,), jnp.float32),
            pltpu.VMEM((_M,), jnp.int32),
            pltpu.VMEM((_K,), jnp.int32),
            pltpu.VMEM((_K,), jnp.int32),
            pltpu.VMEM((_K,), jnp.int32),
            pltpu.VMEM((_CHUNK_WORDS,), jnp.float32),
            pltpu.VMEM((16,), jnp.float32),
            pltpu.SemaphoreType.DMA,
            pltpu.SemaphoreType.DMA,
        ],
        compiler_params=pltpu.CompilerParams(needs_layout_passes=False,
                                             use_tc_tiling_on_sc=False),
    )(_sc_body)
    c_part = sc_fn(gtt, gt_node_corr_overlaps, qr, qs)

    res = pl.pallas_call(
        _tc_combine_body,
        out_shape=jax.ShapeDtypeStruct((1, 8), jnp.float32),
    )(c_part, fine)
    return res[0, :5]


# speedup vs baseline: 2.8925x; 1.0528x over previous
import functools
import math

import jax
import jax.numpy as jnp
from jax import lax
from jax.experimental import pallas as pl
from jax.experimental.pallas import tpu as pltpu
from jax.experimental.pallas import tpu_sc as plsc

_NCOLS = 2048
_CHUNK_ROWS = 32
_CHUNK_WORDS = _CHUNK_ROWS * _NCOLS
_NUM_WORKERS = 32
_PPAD = 30720
_M = 8192
_K = 4096


def _sc_body(gtt_h, ovl_h, qr_h, qs_h,
             c_out_h,
             gr_v, gs_v, ovl_v, elin_v, qr_v, qs_v, qlin_v, mapb,
             acc_c_v, sem_b, sem_c):
    wid = lax.axis_index("s") * 2 + lax.axis_index("c")

    cp_gr = pltpu.async_copy(gtt_h.at[0], gr_v, sem_b)
    cp_gs = pltpu.async_copy(gtt_h.at[1], gs_v, sem_b)
    cp_ovl = pltpu.async_copy(ovl_h, ovl_v, sem_b)
    cp_qr = pltpu.async_copy(qr_h, qr_v, sem_c)
    cp_qs = pltpu.async_copy(qs_h, qs_v, sem_c)

    zeros = jnp.zeros((16,), jnp.float32)
    ones = jnp.ones((16,), jnp.float32)
    izeros = jnp.zeros((16,), jnp.int32)
    lane = lax.iota(jnp.int32, 16)
    wnd_u = jnp.uint32(2 * _CHUNK_WORDS)
    wbase = wid * (2 * _CHUNK_WORDS)
    nwords_u = jnp.uint32(_CHUNK_WORDS)

    cp_gr.wait(); cp_gs.wait(); cp_ovl.wait()

    def ebody(i, off):
        for j in range(4):
            o = (i * 4 + j) * 16
            er = gr_v[pl.ds(o, 16)]
            es = gs_v[pl.ds(o, 16)]
            eo = ovl_v[pl.ds(o, 16)]
            rel = er * _NCOLS + es - wbase
            m = (eo > 0.0) & (plsc.bitcast(rel, jnp.uint32) < wnd_u)
            pref = plsc.cumsum(jnp.where(m, 1, 0))
            plsc.store_scatter(elin_v, [off + pref - 1], rel, mask=m)
            off = off + plsc.all_reduce_population_count(m)
        return off

    eoff = lax.fori_loop(0, _M // 64, ebody, izeros)

    cp_qr.wait(); cp_qs.wait()

    def qfbody(i, off):
        for j in range(4):
            o = (i * 4 + j) * 16
            rel = qr_v[pl.ds(o, 16)] * _NCOLS + qs_v[pl.ds(o, 16)] - wbase
            m = plsc.bitcast(rel, jnp.uint32) < wnd_u
            pref = plsc.cumsum(jnp.where(m, 1, 0))
            plsc.store_scatter(qlin_v, [off + pref - 1], rel, mask=m)
            off = off + plsc.all_reduce_population_count(m)
        return off

    qoff = lax.fori_loop(0, _K // 64, qfbody, izeros)

    ecnt = jnp.max(eoff)
    qcnt = jnp.max(qoff)
    ne = (ecnt + 15) // 16
    nq = (qcnt + 15) // 16

    cacc = zeros
    for half in range(2):
        base = half * _CHUNK_WORDS

        def zbody(i, _):
            rel = qlin_v[pl.ds(i * 16, 16)] - base
            m = (i * 16 + lane < qoff) & (plsc.bitcast(rel, jnp.uint32) < nwords_u)
            idx = rel & (_CHUNK_WORDS - 1)
            plsc.store_scatter(mapb, [idx], zeros, mask=m)
            return 0

        lax.fori_loop(0, nq, zbody, 0)

        def sbody(i, _):
            rel = elin_v[pl.ds(i * 16, 16)] - base
            m = (i * 16 + lane < eoff) & (plsc.bitcast(rel, jnp.uint32) < nwords_u)
            idx = rel & (_CHUNK_WORDS - 1)
            plsc.store_scatter(mapb, [idx], ones, mask=m)
            return 0

        lax.fori_loop(0, ne, sbody, 0)

        def gbody(i, acc):
            rel = qlin_v[pl.ds(i * 16, 16)] - base
            m = (i * 16 + lane < qoff) & (plsc.bitcast(rel, jnp.uint32) < nwords_u)
            idx = rel & (_CHUNK_WORDS - 1)
            v = plsc.load_gather(mapb, [idx], mask=m)
            return acc + jnp.where(m, v, 0.0)

        cacc = lax.fori_loop(0, nq, gbody, cacc)

    acc_c_v[...] = cacc
    pltpu.sync_copy(acc_c_v, c_out_h.at[wid])


def _tc_fine_body(prt_ref, pst_ref, t_ref, e_ref, out_ref):
    t = t_ref[...]
    e = e_ref[...]
    pr = prt_ref[...]
    ps = pst_ref[...]
    sx = ps[0:1, :]
    sy = ps[1:2, :]
    sz = ps[2:3, :]
    dx = pr[0:1, :] - (t[0, 0] * sx + t[0, 1] * sy + t[0, 2] * sz + t[0, 3])
    dy = pr[1:2, :] - (t[1, 0] * sx + t[1, 1] * sy + t[1, 2] * sz + t[1, 3])
    dz = pr[2:3, :] - (t[2, 0] * sx + t[2, 1] * sy + t[2, 2] * sz + t[2, 3])
    d2 = dx * dx + dy * dy + dz * dz
    pidx = lax.broadcasted_iota(jnp.int32, (1, _PPAD), 1)
    inlier = (pidx < 30000) & (d2 < 0.01)
    f_prec = jnp.sum(jnp.where(inlier, 1.0, 0.0)) * (1.0 / 30000.0)
    tr = jnp.sum(t[:3, :3] * e[:3, :3])
    x = jnp.clip((tr - 1.0) * 0.5, -1.0, 1.0)
    acos = jnp.arctan2(jnp.sqrt(jnp.maximum(1.0 - x * x, 0.0)), x)
    rre = acos * (180.0 / math.pi)
    dt = t[:3, 3] - e[:3, 3]
    rte = jnp.sqrt(jnp.sum(dt * dt))
    recall = jnp.where((rre < 15.0) & (rte < 0.3), 1.0, 0.0)
    i8 = lax.broadcasted_iota(jnp.int32, (1, 8), 1)
    v = jnp.where(i8 == 1, f_prec,
        jnp.where(i8 == 2, rre,
        jnp.where(i8 == 3, rte,
        jnp.where(i8 == 4, recall, 0.0))))
    out_ref[...] = v


def _tc_combine_body(cpart_ref, fine_ref, out_ref):
    c_prec = jnp.sum(cpart_ref[...]) * (1.0 / 4096.0)
    i8 = lax.broadcasted_iota(jnp.int32, (1, 8), 1)
    out_ref[...] = jnp.where(i8 == 0, c_prec, fine_ref[...])


def kernel(ref_points_c, src_points_c, gt_node_corr_overlaps, gt_node_corr_indices,
           ref_node_corr_indices, src_node_corr_indices, ref_corr_points,
           src_corr_points, transform, estimated_transform):
    gtt = gt_node_corr_indices.astype(jnp.int32).T
    qr = ref_node_corr_indices.astype(jnp.int32)
    qs = src_node_corr_indices.astype(jnp.int32)
    p = ref_corr_points.shape[0]
    prt = jnp.pad(ref_corr_points.T, ((0, 0), (0, _PPAD - p)))
    pst = jnp.pad(src_corr_points.T, ((0, 0), (0, _PPAD - p)))

    fine = pl.pallas_call(
        _tc_fine_body,
        out_shape=jax.ShapeDtypeStruct((1, 8), jnp.float32),
    )(prt, pst, transform.astype(jnp.float32),
      estimated_transform.astype(jnp.float32))

    mesh = plsc.VectorSubcoreMesh(core_axis_name="c", subcore_axis_name="s",
                                  num_cores=2, num_subcores=16)
    sc_fn = functools.partial(
        pl.kernel,
        out_type=jax.ShapeDtypeStruct((_NUM_WORKERS, 16), jnp.float32),
        mesh=mesh,
        scratch_types=[
            pltpu.VMEM((_M,), jnp.int32),
            pltpu.VMEM((_M,), jnp.int32),
            pltpu.VMEM((_M,), jnp.float32),
            pltpu.VMEM((_M,), jnp.int32),
            pltpu.VMEM((_K,), jnp.int32),
            pltpu.VMEM((_K,), jnp.int32),
            pltpu.VMEM((_K,), jnp.int32),
            pltpu.VMEM((_CHUNK_WORDS,), jnp.float32),
            pltpu.VMEM((16,), jnp.float32),
            pltpu.SemaphoreType.DMA,
            pltpu.SemaphoreType.DMA,
        ],
        compiler_params=pltpu.CompilerParams(needs_layout_passes=False,
                                             use_tc_tiling_on_sc=False),
    )(_sc_body)
    c_part = sc_fn(gtt, gt_node_corr_overlaps, qr, qs)

    res = pl.pallas_call(
        _tc_combine_body,
        out_shape=jax.ShapeDtypeStruct((1, 8), jnp.float32),
    )(c_part, fine)
    return res[0, :5]


# speedup vs baseline: 3.0869x; 1.0672x over previous
import functools
import math

import jax
import jax.numpy as jnp
from jax import lax
from jax.experimental import pallas as pl
from jax.experimental.pallas import tpu as pltpu
from jax.experimental.pallas import tpu_sc as plsc

_NCOLS = 2048
_CHUNK_ROWS = 32
_CHUNK_WORDS = _CHUNK_ROWS * _NCOLS
_NUM_WORKERS = 32
_PPAD = 30720
_M = 8192
_K = 4096


def _sc_body(gtt_h, ovl_h, qr_h, qs_h,
             c_out_h,
             gr_v, gs_v, ovl_v, elin_v, qr_v, qs_v, qlin_v, mapb,
             acc_c_v, sem_b, sem_c):
    wid = lax.axis_index("s") * 2 + lax.axis_index("c")

    cp_gr = pltpu.async_copy(gtt_h.at[0], gr_v, sem_b)
    cp_gs = pltpu.async_copy(gtt_h.at[1], gs_v, sem_b)
    cp_ovl = pltpu.async_copy(ovl_h, ovl_v, sem_b)
    cp_qr = pltpu.async_copy(qr_h, qr_v, sem_c)
    cp_qs = pltpu.async_copy(qs_h, qs_v, sem_c)

    zeros = jnp.zeros((16,), jnp.float32)
    ones = jnp.ones((16,), jnp.float32)
    izeros = jnp.zeros((16,), jnp.int32)
    lane = lax.iota(jnp.int32, 16)
    wnd_u = jnp.uint32(2 * _CHUNK_WORDS)
    wbase = wid * (2 * _CHUNK_WORDS)
    nwords_u = jnp.uint32(_CHUNK_WORDS)

    lane_e = lane * (_M // 16)
    lane_q = lane * (_K // 16)
    cp_gr.wait(); cp_gs.wait(); cp_ovl.wait()

    def ebody(i, off):
        for j in range(4):
            o = (i * 4 + j) * 16
            er = gr_v[pl.ds(o, 16)]
            es = gs_v[pl.ds(o, 16)]
            eo = ovl_v[pl.ds(o, 16)]
            rel = er * _NCOLS + es - wbase
            m = (eo > 0.0) & (plsc.bitcast(rel, jnp.uint32) < wnd_u)
            plsc.store_scatter(elin_v, [lane_e + off], rel, mask=m)
            off = off + jnp.where(m, 1, 0)
        return off

    eoff = lax.fori_loop(0, _M // 64, ebody, izeros)

    cp_qr.wait(); cp_qs.wait()

    def qfbody(i, off):
        for j in range(4):
            o = (i * 4 + j) * 16
            rel = qr_v[pl.ds(o, 16)] * _NCOLS + qs_v[pl.ds(o, 16)] - wbase
            m = plsc.bitcast(rel, jnp.uint32) < wnd_u
            plsc.store_scatter(qlin_v, [lane_q + off], rel, mask=m)
            off = off + jnp.where(m, 1, 0)
        return off

    qoff = lax.fori_loop(0, _K // 64, qfbody, izeros)

    ne = jnp.max(eoff)
    nq = jnp.max(qoff)

    cacc = zeros
    for half in range(2):
        base = half * _CHUNK_WORDS

        def zbody(i, _):
            rel = plsc.load_gather(qlin_v, [lane_q + i]) - base
            m = (i < qoff) & (plsc.bitcast(rel, jnp.uint32) < nwords_u)
            idx = rel & (_CHUNK_WORDS - 1)
            plsc.store_scatter(mapb, [idx], zeros, mask=m)
            return 0

        lax.fori_loop(0, nq, zbody, 0)

        def sbody(i, _):
            rel = plsc.load_gather(elin_v, [lane_e + i]) - base
            m = (i < eoff) & (plsc.bitcast(rel, jnp.uint32) < nwords_u)
            idx = rel & (_CHUNK_WORDS - 1)
            plsc.store_scatter(mapb, [idx], ones, mask=m)
            return 0

        lax.fori_loop(0, ne, sbody, 0)

        def gbody(i, acc):
            rel = plsc.load_gather(qlin_v, [lane_q + i]) - base
            m = (i < qoff) & (plsc.bitcast(rel, jnp.uint32) < nwords_u)
            idx = rel & (_CHUNK_WORDS - 1)
            v = plsc.load_gather(mapb, [idx], mask=m)
            return acc + jnp.where(m, v, 0.0)

        cacc = lax.fori_loop(0, nq, gbody, cacc)

    acc_c_v[...] = cacc
    pltpu.sync_copy(acc_c_v, c_out_h.at[wid])


def _tc_fine_body(prt_ref, pst_ref, t_ref, e_ref, out_ref):
    t = t_ref[...]
    e = e_ref[...]
    pr = prt_ref[...]
    ps = pst_ref[...]
    sx = ps[0:1, :]
    sy = ps[1:2, :]
    sz = ps[2:3, :]
    dx = pr[0:1, :] - (t[0, 0] * sx + t[0, 1] * sy + t[0, 2] * sz + t[0, 3])
    dy = pr[1:2, :] - (t[1, 0] * sx + t[1, 1] * sy + t[1, 2] * sz + t[1, 3])
    dz = pr[2:3, :] - (t[2, 0] * sx + t[2, 1] * sy + t[2, 2] * sz + t[2, 3])
    d2 = dx * dx + dy * dy + dz * dz
    pidx = lax.broadcasted_iota(jnp.int32, (1, _PPAD), 1)
    inlier = (pidx < 30000) & (d2 < 0.01)
    f_prec = jnp.sum(jnp.where(inlier, 1.0, 0.0)) * (1.0 / 30000.0)
    tr = jnp.sum(t[:3, :3] * e[:3, :3])
    x = jnp.clip((tr - 1.0) * 0.5, -1.0, 1.0)
    acos = jnp.arctan2(jnp.sqrt(jnp.maximum(1.0 - x * x, 0.0)), x)
    rre = acos * (180.0 / math.pi)
    dt = t[:3, 3] - e[:3, 3]
    rte = jnp.sqrt(jnp.sum(dt * dt))
    recall = jnp.where((rre < 15.0) & (rte < 0.3), 1.0, 0.0)
    i8 = lax.broadcasted_iota(jnp.int32, (1, 8), 1)
    v = jnp.where(i8 == 1, f_prec,
        jnp.where(i8 == 2, rre,
        jnp.where(i8 == 3, rte,
        jnp.where(i8 == 4, recall, 0.0))))
    out_ref[...] = v


def _tc_combine_body(cpart_ref, fine_ref, out_ref):
    c_prec = jnp.sum(cpart_ref[...]) * (1.0 / 4096.0)
    i8 = lax.broadcasted_iota(jnp.int32, (1, 8), 1)
    out_ref[...] = jnp.where(i8 == 0, c_prec, fine_ref[...])


def kernel(ref_points_c, src_points_c, gt_node_corr_overlaps, gt_node_corr_indices,
           ref_node_corr_indices, src_node_corr_indices, ref_corr_points,
           src_corr_points, transform, estimated_transform):
    gtt = gt_node_corr_indices.astype(jnp.int32).T
    qr = ref_node_corr_indices.astype(jnp.int32)
    qs = src_node_corr_indices.astype(jnp.int32)
    p = ref_corr_points.shape[0]
    prt = jnp.pad(ref_corr_points.T, ((0, 0), (0, _PPAD - p)))
    pst = jnp.pad(src_corr_points.T, ((0, 0), (0, _PPAD - p)))

    fine = pl.pallas_call(
        _tc_fine_body,
        out_shape=jax.ShapeDtypeStruct((1, 8), jnp.float32),
    )(prt, pst, transform.astype(jnp.float32),
      estimated_transform.astype(jnp.float32))

    mesh = plsc.VectorSubcoreMesh(core_axis_name="c", subcore_axis_name="s",
                                  num_cores=2, num_subcores=16)
    sc_fn = functools.partial(
        pl.kernel,
        out_type=jax.ShapeDtypeStruct((_NUM_WORKERS, 16), jnp.float32),
        mesh=mesh,
        scratch_types=[
            pltpu.VMEM((_M,), jnp.int32),
            pltpu.VMEM((_M,), jnp.int32),
            pltpu.VMEM((_M,), jnp.float32),
            pltpu.VMEM((_M,), jnp.int32),
            pltpu.VMEM((_K,), jnp.int32),
            pltpu.VMEM((_K,), jnp.int32),
            pltpu.VMEM((_K,), jnp.int32),
            pltpu.VMEM((_CHUNK_WORDS,), jnp.float32),
            pltpu.VMEM((16,), jnp.float32),
            pltpu.SemaphoreType.DMA,
            pltpu.SemaphoreType.DMA,
        ],
        compiler_params=pltpu.CompilerParams(needs_layout_passes=False,
                                             use_tc_tiling_on_sc=False),
    )(_sc_body)
    c_part = sc_fn(gtt, gt_node_corr_overlaps, qr, qs)

    res = pl.pallas_call(
        _tc_combine_body,
        out_shape=jax.ShapeDtypeStruct((1, 8), jnp.float32),
    )(c_part, fine)
    return res[0, :5]


# speedup vs baseline: 3.3314x; 1.0792x over previous
import functools
import math

import jax
import jax.numpy as jnp
from jax import lax
from jax.experimental import pallas as pl
from jax.experimental.pallas import tpu as pltpu
from jax.experimental.pallas import tpu_sc as plsc

_NCOLS = 2048
_CHUNK_ROWS = 32
_CHUNK_WORDS = _CHUNK_ROWS * _NCOLS
_NUM_WORKERS = 32
_PPAD = 30720
_M = 8192
_K = 4096


def _sc_body(gtt_h, ovl_h, qr_h, qs_h,
             c_out_h,
             gr_v, gs_v, ovl_v, elin_v, qr_v, qs_v, qlin_v, mapb,
             acc_c_v, sem_b, sem_c, sem_d):
    wid = lax.axis_index("s") * 2 + lax.axis_index("c")

    half_m = _M // 2
    cp_gr0 = pltpu.async_copy(gtt_h.at[0, pl.ds(0, half_m)],
                              gr_v.at[pl.ds(0, half_m)], sem_b)
    cp_gs0 = pltpu.async_copy(gtt_h.at[1, pl.ds(0, half_m)],
                              gs_v.at[pl.ds(0, half_m)], sem_b)
    cp_ovl0 = pltpu.async_copy(ovl_h.at[pl.ds(0, half_m)],
                               ovl_v.at[pl.ds(0, half_m)], sem_b)
    cp_gr1 = pltpu.async_copy(gtt_h.at[0, pl.ds(half_m, half_m)],
                              gr_v.at[pl.ds(half_m, half_m)], sem_c)
    cp_gs1 = pltpu.async_copy(gtt_h.at[1, pl.ds(half_m, half_m)],
                              gs_v.at[pl.ds(half_m, half_m)], sem_c)
    cp_ovl1 = pltpu.async_copy(ovl_h.at[pl.ds(half_m, half_m)],
                               ovl_v.at[pl.ds(half_m, half_m)], sem_c)
    cp_qr = pltpu.async_copy(qr_h, qr_v, sem_d)
    cp_qs = pltpu.async_copy(qs_h, qs_v, sem_d)

    zeros = jnp.zeros((16,), jnp.float32)
    ones = jnp.ones((16,), jnp.float32)
    izeros = jnp.zeros((16,), jnp.int32)
    lane = lax.iota(jnp.int32, 16)
    wnd_u = jnp.uint32(2 * _CHUNK_WORDS)
    wbase = wid * (2 * _CHUNK_WORDS)
    nwords_u = jnp.uint32(_CHUNK_WORDS)

    lane_e = lane * (_M // 16)
    lane_q = lane * (_K // 16)

    def _efilter(base_o):
        def ebody(i, off):
            for j in range(4):
                o = base_o + (i * 4 + j) * 16
                er = gr_v[pl.ds(o, 16)]
                es = gs_v[pl.ds(o, 16)]
                eo = ovl_v[pl.ds(o, 16)]
                rel = er * _NCOLS + es - wbase
                m = (eo > 0.0) & (plsc.bitcast(rel, jnp.uint32) < wnd_u)
                plsc.store_scatter(elin_v, [lane_e + off], rel, mask=m)
                off = off + jnp.where(m, 1, 0)
            return off
        return ebody

    cp_gr0.wait(); cp_gs0.wait(); cp_ovl0.wait()
    eoff = lax.fori_loop(0, _M // 128, _efilter(0), izeros)
    cp_gr1.wait(); cp_gs1.wait(); cp_ovl1.wait()
    eoff = lax.fori_loop(0, _M // 128, _efilter(half_m), eoff)

    cp_qr.wait(); cp_qs.wait()

    def qfbody(i, off):
        for j in range(4):
            o = (i * 4 + j) * 16
            rel = qr_v[pl.ds(o, 16)] * _NCOLS + qs_v[pl.ds(o, 16)] - wbase
            m = plsc.bitcast(rel, jnp.uint32) < wnd_u
            plsc.store_scatter(qlin_v, [lane_q + off], rel, mask=m)
            off = off + jnp.where(m, 1, 0)
        return off

    qoff = lax.fori_loop(0, _K // 64, qfbody, izeros)

    ne = jnp.max(eoff)
    nq = jnp.max(qoff)

    cacc = zeros
    for half in range(2):
        base = half * _CHUNK_WORDS

        def zbody(i, _):
            rel = plsc.load_gather(qlin_v, [lane_q + i]) - base
            m = (i < qoff) & (plsc.bitcast(rel, jnp.uint32) < nwords_u)
            idx = rel & (_CHUNK_WORDS - 1)
            plsc.store_scatter(mapb, [idx], zeros, mask=m)
            return 0

        lax.fori_loop(0, nq, zbody, 0)

        def sbody(i, _):
            rel = plsc.load_gather(elin_v, [lane_e + i]) - base
            m = (i < eoff) & (plsc.bitcast(rel, jnp.uint32) < nwords_u)
            idx = rel & (_CHUNK_WORDS - 1)
            plsc.store_scatter(mapb, [idx], ones, mask=m)
            return 0

        lax.fori_loop(0, ne, sbody, 0)

        def gbody(i, acc):
            rel = plsc.load_gather(qlin_v, [lane_q + i]) - base
            m = (i < qoff) & (plsc.bitcast(rel, jnp.uint32) < nwords_u)
            idx = rel & (_CHUNK_WORDS - 1)
            v = plsc.load_gather(mapb, [idx], mask=m)
            return acc + jnp.where(m, v, 0.0)

        cacc = lax.fori_loop(0, nq, gbody, cacc)

    acc_c_v[...] = cacc
    pltpu.sync_copy(acc_c_v, c_out_h.at[pl.ds(wid * 16, 16)])


def _tc_fine_body(prt_ref, pst_ref, t_ref, e_ref, out_ref):
    t = t_ref[...]
    e = e_ref[...]
    pr = prt_ref[...]
    ps = pst_ref[...]
    sx = ps[0:1, :]
    sy = ps[1:2, :]
    sz = ps[2:3, :]
    dx = pr[0:1, :] - (t[0, 0] * sx + t[0, 1] * sy + t[0, 2] * sz + t[0, 3])
    dy = pr[1:2, :] - (t[1, 0] * sx + t[1, 1] * sy + t[1, 2] * sz + t[1, 3])
    dz = pr[2:3, :] - (t[2, 0] * sx + t[2, 1] * sy + t[2, 2] * sz + t[2, 3])
    d2 = dx * dx + dy * dy + dz * dz
    pidx = lax.broadcasted_iota(jnp.int32, (1, _PPAD), 1)
    inlier = (pidx < 30000) & (d2 < 0.01)
    f_prec = jnp.sum(jnp.where(inlier, 1.0, 0.0)) * (1.0 / 30000.0)
    tr = jnp.sum(t[:3, :3] * e[:3, :3])
    x = jnp.clip((tr - 1.0) * 0.5, -1.0, 1.0)
    acos = jnp.arctan2(jnp.sqrt(jnp.maximum(1.0 - x * x, 0.0)), x)
    rre = acos * (180.0 / math.pi)
    dt = t[:3, 3] - e[:3, 3]
    rte = jnp.sqrt(jnp.sum(dt * dt))
    recall = jnp.where((rre < 15.0) & (rte < 0.3), 1.0, 0.0)
    i8 = lax.broadcasted_iota(jnp.int32, (1, 8), 1)
    v = jnp.where(i8 == 1, f_prec,
        jnp.where(i8 == 2, rre,
        jnp.where(i8 == 3, rte,
        jnp.where(i8 == 4, recall, 0.0))))
    out_ref[...] = v


def _tc_combine_body(cpart_ref, fine_ref, out_ref):
    c_prec = jnp.sum(cpart_ref[...]) * (1.0 / 4096.0)
    i5 = lax.broadcasted_iota(jnp.int32, (5,), 0)
    out_ref[...] = jnp.where(i5 == 0, c_prec, fine_ref[0, :5])


def kernel(ref_points_c, src_points_c, gt_node_corr_overlaps, gt_node_corr_indices,
           ref_node_corr_indices, src_node_corr_indices, ref_corr_points,
           src_corr_points, transform, estimated_transform):
    gtt = gt_node_corr_indices.astype(jnp.int32).T
    qr = ref_node_corr_indices.astype(jnp.int32)
    qs = src_node_corr_indices.astype(jnp.int32)
    p = ref_corr_points.shape[0]
    prt = jnp.pad(ref_corr_points.T, ((0, 0), (0, _PPAD - p)))
    pst = jnp.pad(src_corr_points.T, ((0, 0), (0, _PPAD - p)))

    fine = pl.pallas_call(
        _tc_fine_body,
        out_shape=jax.ShapeDtypeStruct((1, 8), jnp.float32),
    )(prt, pst, transform.astype(jnp.float32),
      estimated_transform.astype(jnp.float32))

    mesh = plsc.VectorSubcoreMesh(core_axis_name="c", subcore_axis_name="s",
                                  num_cores=2, num_subcores=16)
    sc_fn = functools.partial(
        pl.kernel,
        out_type=jax.ShapeDtypeStruct((_NUM_WORKERS * 16,), jnp.float32),
        mesh=mesh,
        scratch_types=[
            pltpu.VMEM((_M,), jnp.int32),
            pltpu.VMEM((_M,), jnp.int32),
            pltpu.VMEM((_M,), jnp.float32),
            pltpu.VMEM((_M,), jnp.int32),
            pltpu.VMEM((_K,), jnp.int32),
            pltpu.VMEM((_K,), jnp.int32),
            pltpu.VMEM((_K,), jnp.int32),
            pltpu.VMEM((_CHUNK_WORDS,), jnp.float32),
            pltpu.VMEM((16,), jnp.float32),
            pltpu.SemaphoreType.DMA,
            pltpu.SemaphoreType.DMA,
            pltpu.SemaphoreType.DMA,
        ],
        compiler_params=pltpu.CompilerParams(needs_layout_passes=False,
                                             use_tc_tiling_on_sc=False),
    )(_sc_body)
    c_part = sc_fn(gtt, gt_node_corr_overlaps, qr, qs)

    res = pl.pallas_call(
        _tc_combine_body,
        out_shape=jax.ShapeDtypeStruct((5,), jnp.float32),
    )(c_part, fine)
    return res


# speedup vs baseline: 3.8258x; 1.1484x over previous
import functools
import math

import jax
import jax.numpy as jnp
from jax import lax
from jax.experimental import pallas as pl
from jax.experimental.pallas import tpu as pltpu
from jax.experimental.pallas import tpu_sc as plsc

_NCOLS = 2048
_CHUNK_ROWS = 32
_CHUNK_WORDS = _CHUNK_ROWS * _NCOLS
_NUM_WORKERS = 32
_PPAD = 30720
_M = 8192
_K = 4096


def _sc_body(gtt_h, ovl_h, qr_h, qs_h,
             c_out_h,
             gr_v, gs_v, ovl_v, elin_v, qr_v, qs_v, qlin_v, mapb,
             acc_c_v, sem_b, sem_c, sem_d):
    wid = lax.axis_index("s") * 2 + lax.axis_index("c")

    half_m = _M // 2
    cp_gr0 = pltpu.async_copy(gtt_h.at[0, pl.ds(0, half_m)],
                              gr_v.at[pl.ds(0, half_m)], sem_b)
    cp_gs0 = pltpu.async_copy(gtt_h.at[1, pl.ds(0, half_m)],
                              gs_v.at[pl.ds(0, half_m)], sem_b)
    cp_ovl0 = pltpu.async_copy(ovl_h.at[pl.ds(0, half_m)],
                               ovl_v.at[pl.ds(0, half_m)], sem_b)
    cp_gr1 = pltpu.async_copy(gtt_h.at[0, pl.ds(half_m, half_m)],
                              gr_v.at[pl.ds(half_m, half_m)], sem_c)
    cp_gs1 = pltpu.async_copy(gtt_h.at[1, pl.ds(half_m, half_m)],
                              gs_v.at[pl.ds(half_m, half_m)], sem_c)
    cp_ovl1 = pltpu.async_copy(ovl_h.at[pl.ds(half_m, half_m)],
                               ovl_v.at[pl.ds(half_m, half_m)], sem_c)
    cp_qr = pltpu.async_copy(qr_h, qr_v, sem_d)
    cp_qs = pltpu.async_copy(qs_h, qs_v, sem_d)

    zeros = jnp.zeros((16,), jnp.float32)
    ones = jnp.ones((16,), jnp.float32)
    izeros = jnp.zeros((16,), jnp.int32)
    lane = lax.iota(jnp.int32, 16)
    wnd_u = jnp.uint32(2 * _CHUNK_WORDS)
    wbase = wid * (2 * _CHUNK_WORDS)
    nwords_u = jnp.uint32(_CHUNK_WORDS)

    lane_e = lane * (_M // 16)
    lane_q = lane * (_K // 16)

    _UNROLL = 8

    def _efilter(base_o):
        def ebody(i, off):
            rels, ms = [], []
            for j in range(_UNROLL):
                o = base_o + (i * _UNROLL + j) * 16
                er = gr_v[pl.ds(o, 16)]
                es = gs_v[pl.ds(o, 16)]
                eo = ovl_v[pl.ds(o, 16)]
                rel = er * _NCOLS + es - wbase
                ms.append((eo > 0.0) & (plsc.bitcast(rel, jnp.uint32) < wnd_u))
                rels.append(rel)
            offs = []
            for j in range(_UNROLL):
                offs.append(off)
                off = off + jnp.where(ms[j], 1, 0)
            for j in range(_UNROLL):
                plsc.store_scatter(elin_v, [lane_e + offs[j]], rels[j], mask=ms[j])
            return off
        return ebody

    cp_gr0.wait(); cp_gs0.wait(); cp_ovl0.wait()
    eoff = lax.fori_loop(0, _M // (32 * _UNROLL), _efilter(0), izeros)
    cp_gr1.wait(); cp_gs1.wait(); cp_ovl1.wait()
    eoff = lax.fori_loop(0, _M // (32 * _UNROLL), _efilter(half_m), eoff)

    cp_qr.wait(); cp_qs.wait()

    def qfbody(i, off):
        rels, ms = [], []
        for j in range(_UNROLL):
            o = (i * _UNROLL + j) * 16
            rel = qr_v[pl.ds(o, 16)] * _NCOLS + qs_v[pl.ds(o, 16)] - wbase
            ms.append(plsc.bitcast(rel, jnp.uint32) < wnd_u)
            rels.append(rel)
        offs = []
        for j in range(_UNROLL):
            offs.append(off)
            off = off + jnp.where(ms[j], 1, 0)
        for j in range(_UNROLL):
            plsc.store_scatter(qlin_v, [lane_q + offs[j]], rels[j], mask=ms[j])
        return off

    qoff = lax.fori_loop(0, _K // (16 * _UNROLL), qfbody, izeros)

    ne = jnp.max(eoff)
    nq = jnp.max(qoff)

    cacc = zeros
    for half in range(2):
        base = half * _CHUNK_WORDS

        def zbody(i, _):
            rel = plsc.load_gather(qlin_v, [lane_q + i]) - base
            m = (i < qoff) & (plsc.bitcast(rel, jnp.uint32) < nwords_u)
            idx = rel & (_CHUNK_WORDS - 1)
            plsc.store_scatter(mapb, [idx], zeros, mask=m)
            return 0

        lax.fori_loop(0, nq, zbody, 0)

        def sbody(i, _):
            rel = plsc.load_gather(elin_v, [lane_e + i]) - base
            m = (i < eoff) & (plsc.bitcast(rel, jnp.uint32) < nwords_u)
            idx = rel & (_CHUNK_WORDS - 1)
            plsc.store_scatter(mapb, [idx], ones, mask=m)
            return 0

        lax.fori_loop(0, ne, sbody, 0)

        def gbody(i, acc):
            rel = plsc.load_gather(qlin_v, [lane_q + i]) - base
            m = (i < qoff) & (plsc.bitcast(rel, jnp.uint32) < nwords_u)
            idx = rel & (_CHUNK_WORDS - 1)
            v = plsc.load_gather(mapb, [idx], mask=m)
            return acc + jnp.where(m, v, 0.0)

        cacc = lax.fori_loop(0, nq, gbody, cacc)

    acc_c_v[...] = cacc
    pltpu.sync_copy(acc_c_v, c_out_h.at[pl.ds(wid * 16, 16)])


def _tc_fine_body(prt_ref, pst_ref, t_ref, e_ref, out_ref):
    t = t_ref[...]
    e = e_ref[...]
    pr = prt_ref[...]
    ps = pst_ref[...]
    sx = ps[0:1, :]
    sy = ps[1:2, :]
    sz = ps[2:3, :]
    dx = pr[0:1, :] - (t[0, 0] * sx + t[0, 1] * sy + t[0, 2] * sz + t[0, 3])
    dy = pr[1:2, :] - (t[1, 0] * sx + t[1, 1] * sy + t[1, 2] * sz + t[1, 3])
    dz = pr[2:3, :] - (t[2, 0] * sx + t[2, 1] * sy + t[2, 2] * sz + t[2, 3])
    d2 = dx * dx + dy * dy + dz * dz
    pidx = lax.broadcasted_iota(jnp.int32, (1, _PPAD), 1)
    inlier = (pidx < 30000) & (d2 < 0.01)
    f_prec = jnp.sum(jnp.where(inlier, 1.0, 0.0)) * (1.0 / 30000.0)
    tr = jnp.sum(t[:3, :3] * e[:3, :3])
    x = jnp.clip((tr - 1.0) * 0.5, -1.0, 1.0)
    acos = jnp.arctan2(jnp.sqrt(jnp.maximum(1.0 - x * x, 0.0)), x)
    rre = acos * (180.0 / math.pi)
    dt = t[:3, 3] - e[:3, 3]
    rte = jnp.sqrt(jnp.sum(dt * dt))
    recall = jnp.where((rre < 15.0) & (rte < 0.3), 1.0, 0.0)
    i8 = lax.broadcasted_iota(jnp.int32, (1, 8), 1)
    v = jnp.where(i8 == 1, f_prec,
        jnp.where(i8 == 2, rre,
        jnp.where(i8 == 3, rte,
        jnp.where(i8 == 4, recall, 0.0))))
    out_ref[...] = v


def _tc_combine_body(cpart_ref, fine_ref, out_ref):
    c_prec = jnp.sum(cpart_ref[...]) * (1.0 / 4096.0)
    i5 = lax.broadcasted_iota(jnp.int32, (5,), 0)
    out_ref[...] = jnp.where(i5 == 0, c_prec, fine_ref[0, :5])


def kernel(ref_points_c, src_points_c, gt_node_corr_overlaps, gt_node_corr_indices,
           ref_node_corr_indices, src_node_corr_indices, ref_corr_points,
           src_corr_points, transform, estimated_transform):
    gtt = gt_node_corr_indices.astype(jnp.int32).T
    qr = ref_node_corr_indices.astype(jnp.int32)
    qs = src_node_corr_indices.astype(jnp.int32)
    p = ref_corr_points.shape[0]
    prt = jnp.pad(ref_corr_points.T, ((0, 0), (0, _PPAD - p)))
    pst = jnp.pad(src_corr_points.T, ((0, 0), (0, _PPAD - p)))

    fine = pl.pallas_call(
        _tc_fine_body,
        out_shape=jax.ShapeDtypeStruct((1, 8), jnp.float32),
    )(prt, pst, transform.astype(jnp.float32),
      estimated_transform.astype(jnp.float32))

    mesh = plsc.VectorSubcoreMesh(core_axis_name="c", subcore_axis_name="s",
                                  num_cores=2, num_subcores=16)
    sc_fn = functools.partial(
        pl.kernel,
        out_type=jax.ShapeDtypeStruct((_NUM_WORKERS * 16,), jnp.float32),
        mesh=mesh,
        scratch_types=[
            pltpu.VMEM((_M,), jnp.int32),
            pltpu.VMEM((_M,), jnp.int32),
            pltpu.VMEM((_M,), jnp.float32),
            pltpu.VMEM((_M,), jnp.int32),
            pltpu.VMEM((_K,), jnp.int32),
            pltpu.VMEM((_K,), jnp.int32),
            pltpu.VMEM((_K,), jnp.int32),
            pltpu.VMEM((_CHUNK_WORDS,), jnp.float32),
            pltpu.VMEM((16,), jnp.float32),
            pltpu.SemaphoreType.DMA,
            pltpu.SemaphoreType.DMA,
            pltpu.SemaphoreType.DMA,
        ],
        compiler_params=pltpu.CompilerParams(needs_layout_passes=False,
                                             use_tc_tiling_on_sc=False),
    )(_sc_body)
    c_part = sc_fn(gtt, gt_node_corr_overlaps, qr, qs)

    res = pl.pallas_call(
        _tc_combine_body,
        out_shape=jax.ShapeDtypeStruct((5,), jnp.float32),
    )(c_part, fine)
    return res
